# Initial kernel scaffold; baseline (speedup 1.0000x reference)
import jax
import jax.numpy as jnp
from jax.experimental import pallas as pl


def kernel(x, edge_index, num_nodes, W, b, bn_w, bn_b):
    raise NotImplementedError("write your pallas kernel here")


# speedup vs baseline: 15.4652x; 15.4652x over previous
import functools

import jax
import jax.numpy as jnp
from jax import lax
from jax.experimental import pallas as pl
from jax.experimental.pallas import tpu as pltpu
from jax.experimental.pallas import tpu_sc as plsc

N = 10000
E = 320000
D = 128
EPS = 1e-5

NC = 2
NS = 16
NW = NC * NS
L = 16
EPW = E // NW
CH = 80
NCHUNK = EPW // CH
NP = 10240
STRIPE = NP // NS
TRASH = N

_mesh = plsc.VectorSubcoreMesh(core_axis_name="c", subcore_axis_name="s")


@functools.partial(
    pl.kernel,
    out_type=jax.ShapeDtypeStruct((NC, NP), jnp.float32),
    mesh=_mesh,
    scratch_types=[
        pltpu.VMEM((CH,), jnp.int32),
        pltpu.VMEM((CH,), jnp.int32),
        pltpu.VMEM((CH,), jnp.int32),
        pltpu.VMEM((CH,), jnp.float32),
        pltpu.VMEM_SHARED((NP,), jnp.float32),
    ],
)
def _deg_sc(src_hbm, dst_hbm, zero_hbm, deg_out, src_v, dst_v, dstp_v,
            ones_v, deg_sh):
    c = lax.axis_index("c")
    s = lax.axis_index("s")
    wid = s * NC + c

    def fill16(i, _):
        ones_v[pl.ds(i * L, L)] = jnp.full((L,), 1.0, jnp.float32)
        return 0

    lax.fori_loop(0, CH // L, fill16, 0)
    pltpu.sync_copy(zero_hbm.at[pl.ds(s * STRIPE, STRIPE)],
                    deg_sh.at[pl.ds(s * STRIPE, STRIPE)])
    plsc.subcore_barrier()

    def step(k, _):
        base = wid * EPW + k * CH
        pltpu.sync_copy(src_hbm.at[pl.ds(base, CH)], src_v)
        pltpu.sync_copy(dst_hbm.at[pl.ds(base, CH)], dst_v)
        for j in range(CH // L):
            sv = src_v[pl.ds(j * L, L)]
            dv = dst_v[pl.ds(j * L, L)]
            dstp_v[pl.ds(j * L, L)] = jnp.where(
                sv == dv, jnp.full((L,), TRASH, jnp.int32), dv)
        pltpu.sync_copy(ones_v, deg_sh.at[dstp_v], add=True)
        return 0

    lax.fori_loop(0, NCHUNK, step, 0)
    plsc.subcore_barrier()
    pltpu.sync_copy(deg_sh.at[pl.ds(s * STRIPE, STRIPE)],
                    deg_out.at[c, pl.ds(s * STRIPE, STRIPE)])


def _tc1_body(x_ref, w_ref, d0_ref, d1_ref, hs_ref, dinv_ref):
    deg = d0_ref[...] + d1_ref[...] + 1.0
    dinv = lax.rsqrt(deg)
    h = lax.dot_general(x_ref[...], w_ref[...],
                        dimension_numbers=(((1,), (1,)), ((), ())),
                        preferred_element_type=jnp.float32)
    hs_ref[...] = h * dinv
    dinv_ref[...] = dinv


def _tc1(x, w, d0, d1):
    return pl.pallas_call(
        _tc1_body,
        out_shape=(jax.ShapeDtypeStruct((N, D), jnp.float32),
                   jax.ShapeDtypeStruct((N, 1), jnp.float32)),
    )(x, w, d0, d1)


@functools.partial(
    pl.kernel,
    out_type=jax.ShapeDtypeStruct((NC, NP, D), jnp.float32),
    mesh=_mesh,
    scratch_types=[
        pltpu.VMEM((CH,), jnp.int32),
        pltpu.VMEM((CH,), jnp.int32),
        pltpu.VMEM((CH,), jnp.int32),
        pltpu.VMEM((CH, D), jnp.float32),
        pltpu.VMEM_SHARED((NP, D), jnp.float32),
    ],
)
def _agg_sc(src_hbm, dst_hbm, hs_hbm, zero_hbm, agg_out, src_v, dst_v,
            dstp_v, rows_v, acc_sh):
    c = lax.axis_index("c")
    s = lax.axis_index("s")
    wid = s * NC + c

    pltpu.sync_copy(zero_hbm.at[pl.ds(s * STRIPE, STRIPE)],
                    acc_sh.at[pl.ds(s * STRIPE, STRIPE)])
    plsc.subcore_barrier()

    def step(k, _):
        base = wid * EPW + k * CH
        pltpu.sync_copy(src_hbm.at[pl.ds(base, CH)], src_v)
        pltpu.sync_copy(dst_hbm.at[pl.ds(base, CH)], dst_v)
        for j in range(CH // L):
            sv = src_v[pl.ds(j * L, L)]
            dv = dst_v[pl.ds(j * L, L)]
            dstp_v[pl.ds(j * L, L)] = jnp.where(
                sv == dv, jnp.full((L,), TRASH, jnp.int32), dv)
        pltpu.sync_copy(hs_hbm.at[src_v], rows_v)
        pltpu.sync_copy(rows_v, acc_sh.at[dstp_v], add=True)
        return 0

    lax.fori_loop(0, NCHUNK, step, 0)
    plsc.subcore_barrier()
    pltpu.sync_copy(acc_sh.at[pl.ds(s * STRIPE, STRIPE)],
                    agg_out.at[c, pl.ds(s * STRIPE, STRIPE)])


def _tc2_body(a0_ref, a1_ref, hs_ref, dinv_ref, b_ref, bnw_ref, bnb_ref,
              out_ref):
    t = (a0_ref[...] + a1_ref[...] + hs_ref[...]) * dinv_ref[...] + b_ref[...]
    mean = jnp.mean(t, axis=0, keepdims=True)
    var = jnp.mean((t - mean) ** 2, axis=0, keepdims=True)
    y = (t - mean) * lax.rsqrt(var + EPS) * bnw_ref[...] + bnb_ref[...]
    out_ref[...] = jnp.maximum(y, 0.0)


def _tc2(a0, a1, hs, dinv, b, bn_w, bn_b):
    return pl.pallas_call(
        _tc2_body,
        out_shape=jax.ShapeDtypeStruct((N, D), jnp.float32),
    )(a0, a1, hs, dinv, b, bn_w, bn_b)


def kernel(x, edge_index, num_nodes, W, b, bn_w, bn_b):
    del num_nodes
    src = edge_index[0]
    dst = edge_index[1]
    zero_col = jnp.zeros((NP,), jnp.float32)
    zero_rows = jnp.zeros((NP, D), jnp.float32)

    degp = _deg_sc(src, dst, zero_col)
    d0 = degp[0, :N].reshape(N, 1)
    d1 = degp[1, :N].reshape(N, 1)
    hs, dinv = _tc1(x, W, d0, d1)
    aggp = _agg_sc(src, dst, hs, zero_rows)
    return _tc2(aggp[0, :N], aggp[1, :N], hs, dinv,
                b.reshape(1, D), bn_w.reshape(1, D), bn_b.reshape(1, D))
